# Initial kernel scaffold; baseline (speedup 1.0000x reference)
import jax
import jax.numpy as jnp
from jax.experimental import pallas as pl


def kernel(node_features, ref_node_features, atom_type, edge_features, ref_edge_features, edge_type, mask_to_nrme, mask_to_erme):
    raise NotImplementedError("write your pallas kernel here")


# speedup vs baseline: 5.9214x; 5.9214x over previous
import functools

import jax
import jax.numpy as jnp
from jax import lax
from jax.experimental import pallas as pl

N_ATOM_TYPES = 4
N_BOND_TYPES = 16


def _seg_body(x_ref, r_ref, t_ref, abs_ref, sq_ref, cnt_ref, *, n_types):
    i = pl.program_id(0)
    d = x_ref[...] - r_ref[...]
    t = t_ref[0, 0, :]
    oh = (t[:, None] == lax.broadcasted_iota(jnp.int32, (1, n_types), 1)
          ).astype(jnp.float32)
    dn = (((0,), (0,)), ((), ()))
    a = lax.dot_general(oh, jnp.abs(d), dimension_numbers=dn,
                        preferred_element_type=jnp.float32)
    s = lax.dot_general(oh, d * d, dimension_numbers=dn,
                        preferred_element_type=jnp.float32)
    c = jnp.sum(oh, axis=0).reshape(1, n_types)

    @pl.when(i == 0)
    def _init():
        abs_ref[...] = a
        sq_ref[...] = s
        cnt_ref[...] = c

    @pl.when(i > 0)
    def _acc():
        abs_ref[...] += a
        sq_ref[...] += s
        cnt_ref[...] += c


def _segment_sums(x, r, t, n_types, block_rows):
    n, w = x.shape
    assert n % block_rows == 0
    nb = n // block_rows
    t3 = t.reshape(nb, 1, block_rows)
    return pl.pallas_call(
        functools.partial(_seg_body, n_types=n_types),
        grid=(nb,),
        in_specs=[
            pl.BlockSpec((block_rows, w), lambda i: (i, 0)),
            pl.BlockSpec((block_rows, w), lambda i: (i, 0)),
            pl.BlockSpec((1, 1, block_rows), lambda i: (i, 0, 0)),
        ],
        out_specs=[
            pl.BlockSpec((n_types, w), lambda i: (0, 0)),
            pl.BlockSpec((n_types, w), lambda i: (0, 0)),
            pl.BlockSpec((1, n_types), lambda i: (0, 0)),
        ],
        out_shape=[
            jax.ShapeDtypeStruct((n_types, w), jnp.float32),
            jax.ShapeDtypeStruct((n_types, w), jnp.float32),
            jax.ShapeDtypeStruct((1, n_types), jnp.float32),
        ],
    )(x, r, t3)


def _combine_body(na_ref, ns_ref, nc_ref, ea_ref, es_ref, ec_ref,
                  nm_ref, em_ref, out_ref):
    def part(a, s, c, m):
        cc = jnp.maximum(c, 1.0)[:, None]
        mm = m * (c > 0.0).astype(jnp.float32)[:, None]
        denom = jnp.maximum(jnp.sum(mm), 1.0)
        mean_abs = jnp.sum((a / cc) * mm) / denom
        mean_sq = jnp.sum((s / cc) * mm) / denom
        return 0.5 * (mean_abs + jnp.sqrt(mean_sq))

    onsite = part(na_ref[...], ns_ref[...], nc_ref[0, :], nm_ref[...])
    hopping = part(ea_ref[...], es_ref[...], ec_ref[0, :], em_ref[...])
    out_ref[...] = (0.5 * (onsite + hopping))[None, None]


def kernel(node_features, ref_node_features, atom_type,
           edge_features, ref_edge_features, edge_type,
           mask_to_nrme, mask_to_erme):
    na, ns, nc = _segment_sums(node_features, ref_node_features,
                               atom_type.astype(jnp.int32),
                               N_ATOM_TYPES, 2000)
    ea, es, ec = _segment_sums(edge_features, ref_edge_features,
                               edge_type.astype(jnp.int32),
                               N_BOND_TYPES, 8000)
    nm = mask_to_nrme.astype(jnp.float32)
    em = mask_to_erme.astype(jnp.float32)
    out = pl.pallas_call(
        _combine_body,
        out_shape=jax.ShapeDtypeStruct((1, 1), jnp.float32),
    )(na, ns, nc, ea, es, ec, nm, em)
    return out.reshape(())
